# Initial kernel scaffold; baseline (speedup 1.0000x reference)
import jax
import jax.numpy as jnp
from jax.experimental import pallas as pl


def kernel(x, edge_index, kernel_vals, W1, b1, W2, b2):
    raise NotImplementedError("write your pallas kernel here")


# speedup vs baseline: 2.6292x; 2.6292x over previous
import functools

import jax
import jax.numpy as jnp
from jax import lax
from jax.experimental import pallas as pl
from jax.experimental.pallas import tpu as pltpu
from jax.experimental.pallas import tpu_sc as plsc

N = 10000
S = 128
HID = 16
OUT = 3
C = 640
NTILES = 32
RPT = 4
NVEC = N // 16
GPC = C // 16

_mesh = plsc.VectorSubcoreMesh(core_axis_name="c", subcore_axis_name="s")


def _make_conv(F_in, E_pad):
    npasses = F_in // (NTILES * RPT)
    nchunks = E_pad // C
    npairs = nchunks // 2

    @functools.partial(
        pl.kernel,
        out_type=jax.ShapeDtypeStruct((2 * F_in, N), jnp.float32),
        mesh=_mesh,
        compiler_params=pltpu.CompilerParams(needs_layout_passes=False),
        scratch_types=[
            pltpu.VMEM((RPT, N), jnp.float32),
            pltpu.VMEM((2 * RPT, N), jnp.float32),
            pltpu.VMEM((2, C), jnp.int32),
            pltpu.VMEM((2, C), jnp.int32),
            pltpu.VMEM((2, C), jnp.float32),
            pltpu.VMEM((2, C), jnp.float32),
            pltpu.SemaphoreType.DMA,
            pltpu.SemaphoreType.DMA,
        ],
    )
    def conv(h_hbm, src_hbm, dst_hbm, w0_hbm, w1_hbm, out_hbm,
             rows_v, acc_v, src_v, dst_v, w0_v, w1_v, sem0, sem1):
        cid = lax.axis_index("c")
        sid = lax.axis_index("s")
        wid = sid * 2 + cid

        def start(c, b, sem):
            pltpu.async_copy(src_hbm.at[pl.ds(c * C, C)], src_v.at[b], sem)
            pltpu.async_copy(dst_hbm.at[pl.ds(c * C, C)], dst_v.at[b], sem)
            pltpu.async_copy(w0_hbm.at[pl.ds(c * C, C)], w0_v.at[b], sem)
            pltpu.async_copy(w1_hbm.at[pl.ds(c * C, C)], w1_v.at[b], sem)

        def wait(c, b, sem):
            pltpu.make_async_copy(src_hbm.at[pl.ds(c * C, C)], src_v.at[b], sem).wait()
            pltpu.make_async_copy(dst_hbm.at[pl.ds(c * C, C)], dst_v.at[b], sem).wait()
            pltpu.make_async_copy(w0_hbm.at[pl.ds(c * C, C)], w0_v.at[b], sem).wait()
            pltpu.make_async_copy(w1_hbm.at[pl.ds(c * C, C)], w1_v.at[b], sem).wait()

        def process(b):
            def group(g, _):
                s16 = src_v[b, pl.ds(g * 16, 16)]
                d16 = dst_v[b, pl.ds(g * 16, 16)]
                v0 = w0_v[b, pl.ds(g * 16, 16)]
                v1 = w1_v[b, pl.ds(g * 16, 16)]
                for j in range(RPT):
                    jv = jnp.full((16,), j, jnp.int32)
                    g16 = plsc.load_gather(rows_v, [jv, s16])
                    plsc.addupdate_scatter(
                        acc_v, [jnp.full((16,), j, jnp.int32), d16], g16 * v0)
                    plsc.addupdate_scatter(
                        acc_v, [jnp.full((16,), RPT + j, jnp.int32), d16], g16 * v1)
                return 0

            lax.fori_loop(0, GPC, group, 0)

        for p in range(npasses):
            f_base = p * (NTILES * RPT) + wid * RPT

            for j in range(RPT):
                pltpu.sync_copy(h_hbm.at[f_base + j], rows_v.at[j])

            def zero(i, _):
                z = jnp.zeros((16,), jnp.float32)
                for r in range(2 * RPT):
                    acc_v[r, pl.ds(i * 16, 16)] = z
                return 0

            lax.fori_loop(0, NVEC, zero, 0)

            start(0, 0, sem0)
            start(1, 1, sem1)

            def pair(i, _):
                c0 = 2 * i
                wait(c0, 0, sem0)
                process(0)

                @pl.when(i < npairs - 1)
                def _():
                    start(c0 + 2, 0, sem0)

                wait(c0 + 1, 1, sem1)
                process(1)

                @pl.when(i < npairs - 1)
                def _():
                    start(c0 + 3, 1, sem1)

                return 0

            lax.fori_loop(0, npairs, pair, 0)

            for j in range(RPT):
                pltpu.sync_copy(acc_v.at[j], out_hbm.at[f_base + j])
                pltpu.sync_copy(acc_v.at[RPT + j], out_hbm.at[F_in + f_base + j])

    return conv


NB = 2000


def _mlp_body(x_ref, h1_ref, h2_ref, w1a, w1b, w1c, b1_ref, w2, b2_ref, o_ref):
    acc = jnp.dot(x_ref[...], w1a[...], preferred_element_type=jnp.float32)
    acc = acc + lax.dot_general(
        h1_ref[...], w1b[...], (((0,), (0,)), ((), ())),
        preferred_element_type=jnp.float32)
    acc = acc + lax.dot_general(
        h2_ref[...], w1c[...], (((0,), (0,)), ((), ())),
        preferred_element_type=jnp.float32)
    acc = acc + b1_ref[...]
    acc = jnp.maximum(acc, 0.0)
    o_ref[...] = jnp.dot(acc, w2[...], preferred_element_type=jnp.float32) + b2_ref[...]


_mlp = pl.pallas_call(
    _mlp_body,
    out_shape=jax.ShapeDtypeStruct((N, OUT), jnp.float32),
)


def kernel(x, edge_index, kernel_vals, W1, b1, W2, b2):
    E = edge_index.shape[1]
    pad = (-E) % (2 * C)
    src = edge_index[0].astype(jnp.int32)
    dst = edge_index[1].astype(jnp.int32)
    w0 = kernel_vals[0]
    w1v = kernel_vals[1]
    if pad:
        zi = jnp.zeros((pad,), jnp.int32)
        zf = jnp.zeros((pad,), jnp.float32)
        src = jnp.concatenate([src, zi])
        dst = jnp.concatenate([dst, zi])
        w0 = jnp.concatenate([w0, zf])
        w1v = jnp.concatenate([w1v, zf])
    E_pad = E + pad

    conv1 = _make_conv(S, E_pad)
    conv2 = _make_conv(2 * S, E_pad)

    xT = x.T
    h1T = conv1(xT, src, dst, w0, w1v)
    h2T = conv2(h1T, src, dst, w0, w1v)
    return _mlp(x, h1T, h2T, W1[:S], W1[S:3 * S], W1[3 * S:], b1[None, :],
                W2, b2[None, :])


# speedup vs baseline: 5.1750x; 1.9683x over previous
import functools

import jax
import jax.numpy as jnp
from jax import lax
from jax.experimental import pallas as pl
from jax.experimental.pallas import tpu as pltpu
from jax.experimental.pallas import tpu_sc as plsc

N = 10000
S = 128
HID = 16
OUT = 3
C = 640
NTILES = 32
RPT = 4
NVEC = N // 16
GPC = C // 16

_mesh = plsc.VectorSubcoreMesh(core_axis_name="c", subcore_axis_name="s")


def _make_conv(F_in, E_pad):
    npasses = F_in // (NTILES * RPT)
    nchunks = E_pad // C
    npairs = nchunks // 2

    @functools.partial(
        pl.kernel,
        out_type=jax.ShapeDtypeStruct((2 * F_in, N), jnp.float32),
        mesh=_mesh,
        compiler_params=pltpu.CompilerParams(needs_layout_passes=False),
        scratch_types=[
            pltpu.VMEM((RPT, N), jnp.float32),
            pltpu.VMEM((2 * RPT, N), jnp.float32),
            pltpu.VMEM((2, C), jnp.int32),
            pltpu.VMEM((2, C), jnp.int32),
            pltpu.VMEM((2, C), jnp.float32),
            pltpu.VMEM((2, C), jnp.float32),
            pltpu.SemaphoreType.DMA,
            pltpu.SemaphoreType.DMA,
        ],
    )
    def conv(h_hbm, src_hbm, dst_hbm, w0_hbm, w1_hbm, out_hbm,
             rows_v, acc_v, src_v, dst_v, w0_v, w1_v, sem0, sem1):
        cid = lax.axis_index("c")
        sid = lax.axis_index("s")
        wid = sid * 2 + cid

        def start(c, b, sem):
            pltpu.async_copy(src_hbm.at[pl.ds(c * C, C)], src_v.at[b], sem)
            pltpu.async_copy(dst_hbm.at[pl.ds(c * C, C)], dst_v.at[b], sem)
            pltpu.async_copy(w0_hbm.at[pl.ds(c * C, C)], w0_v.at[b], sem)
            pltpu.async_copy(w1_hbm.at[pl.ds(c * C, C)], w1_v.at[b], sem)

        def wait(c, b, sem):
            pltpu.make_async_copy(src_hbm.at[pl.ds(c * C, C)], src_v.at[b], sem).wait()
            pltpu.make_async_copy(dst_hbm.at[pl.ds(c * C, C)], dst_v.at[b], sem).wait()
            pltpu.make_async_copy(w0_hbm.at[pl.ds(c * C, C)], w0_v.at[b], sem).wait()
            pltpu.make_async_copy(w1_hbm.at[pl.ds(c * C, C)], w1_v.at[b], sem).wait()

        def process(b):
            @plsc.parallel_loop(0, GPC, 1, unroll=4)
            def group(g):
                s16 = src_v[b, pl.ds(g * 16, 16)]
                d16 = dst_v[b, pl.ds(g * 16, 16)]
                v0 = w0_v[b, pl.ds(g * 16, 16)]
                v1 = w1_v[b, pl.ds(g * 16, 16)]
                for j in range(RPT):
                    jv = jnp.full((16,), j, jnp.int32)
                    g16 = plsc.load_gather(rows_v, [jv, s16])
                    plsc.addupdate_scatter(
                        acc_v, [jnp.full((16,), j, jnp.int32), d16], g16 * v0)
                    plsc.addupdate_scatter(
                        acc_v, [jnp.full((16,), RPT + j, jnp.int32), d16], g16 * v1)

        for p in range(npasses):
            f_base = p * (NTILES * RPT) + wid * RPT

            for j in range(RPT):
                pltpu.sync_copy(h_hbm.at[f_base + j], rows_v.at[j])

            @plsc.parallel_loop(0, NVEC, 1, unroll=8)
            def zero(i):
                z = jnp.zeros((16,), jnp.float32)
                for r in range(2 * RPT):
                    acc_v[r, pl.ds(i * 16, 16)] = z

            start(0, 0, sem0)
            start(1, 1, sem1)

            def pair(i, _):
                c0 = 2 * i
                wait(c0, 0, sem0)
                process(0)

                @pl.when(i < npairs - 1)
                def _():
                    start(c0 + 2, 0, sem0)

                wait(c0 + 1, 1, sem1)
                process(1)

                @pl.when(i < npairs - 1)
                def _():
                    start(c0 + 3, 1, sem1)

                return 0

            lax.fori_loop(0, npairs, pair, 0)

            for j in range(RPT):
                pltpu.sync_copy(acc_v.at[j], out_hbm.at[f_base + j])
                pltpu.sync_copy(acc_v.at[RPT + j], out_hbm.at[F_in + f_base + j])

    return conv


NB = 2000


def _mlp_body(x_ref, h1_ref, h2_ref, w1a, w1b, w1c, b1_ref, w2, b2_ref, o_ref):
    acc = jnp.dot(x_ref[...], w1a[...], preferred_element_type=jnp.float32)
    acc = acc + lax.dot_general(
        h1_ref[...], w1b[...], (((0,), (0,)), ((), ())),
        preferred_element_type=jnp.float32)
    acc = acc + lax.dot_general(
        h2_ref[...], w1c[...], (((0,), (0,)), ((), ())),
        preferred_element_type=jnp.float32)
    acc = acc + b1_ref[...]
    acc = jnp.maximum(acc, 0.0)
    o_ref[...] = jnp.dot(acc, w2[...], preferred_element_type=jnp.float32) + b2_ref[...]


_mlp = pl.pallas_call(
    _mlp_body,
    out_shape=jax.ShapeDtypeStruct((N, OUT), jnp.float32),
)


def kernel(x, edge_index, kernel_vals, W1, b1, W2, b2):
    E = edge_index.shape[1]
    pad = (-E) % (2 * C)
    src = edge_index[0].astype(jnp.int32)
    dst = edge_index[1].astype(jnp.int32)
    w0 = kernel_vals[0]
    w1v = kernel_vals[1]
    if pad:
        zi = jnp.zeros((pad,), jnp.int32)
        zf = jnp.zeros((pad,), jnp.float32)
        src = jnp.concatenate([src, zi])
        dst = jnp.concatenate([dst, zi])
        w0 = jnp.concatenate([w0, zf])
        w1v = jnp.concatenate([w1v, zf])
    E_pad = E + pad

    conv1 = _make_conv(S, E_pad)
    conv2 = _make_conv(2 * S, E_pad)

    xT = x.T
    h1T = conv1(xT, src, dst, w0, w1v)
    h2T = conv2(h1T, src, dst, w0, w1v)
    return _mlp(x, h1T, h2T, W1[:S], W1[S:3 * S], W1[3 * S:], b1[None, :],
                W2, b2[None, :])
